# Initial kernel scaffold; baseline (speedup 1.0000x reference)
import jax
import jax.numpy as jnp
from jax.experimental import pallas as pl


def kernel(v_emb, c_emb, edge_index, vmsg_W1, vmsg_b1, vmsg_W2, vmsg_b2, cmsg_W1, cmsg_b1, cmsg_W2, cmsg_b2, vupd_W1, vupd_b1, vupd_W2, vupd_b2, cupd_W1, cupd_b1, cupd_W2, cupd_b2):
    raise NotImplementedError("write your pallas kernel here")


# speedup vs baseline: 5.5963x; 5.5963x over previous
import functools

import jax
import jax.numpy as jnp
from jax import lax
from jax.experimental import pallas as pl
from jax.experimental.pallas import tpu as pltpu
from jax.experimental.pallas import tpu_sc as plsc

NUM_ROUND = 4
N = 10000
E = 320000
D = 128

NS = 16
EPT = E // NS
K = 128
NFULL = EPT // K
KTAIL = EPT - NFULL * K
ROW_GRAN = 200
NCH = N // ROW_GRAN
CH_BASE = NCH // NS
CH_EXTRA = NCH % NS

BR = 2000
G = (2 * N) // BR


def _sc_body(msg_hbm, src_hbm, dst_hbm, dstp_hbm,
             agg_hbm,
             gi_v, si_v, rows_v, gi_t, si_t, rows_t, zbuf, acc, sem):
    cid = lax.axis_index("c")
    sid = lax.axis_index("s")
    n_ch = CH_BASE + jnp.where(sid < CH_EXTRA, 1, 0)
    ch0 = sid * CH_BASE + jnp.minimum(sid, CH_EXTRA)

    def zrow(i, c):
        for j in range(8):
            zbuf[i, pl.ds(j * 16, 16)] = jnp.zeros((16,), jnp.float32)
        return c
    lax.fori_loop(0, ROW_GRAN, zrow, 0)

    def zero_chunk(i, c):
        off = pl.multiple_of((ch0 + i) * ROW_GRAN, 8)
        pltpu.sync_copy(zbuf, acc.at[pl.ds(off, ROW_GRAN)])
        return c
    lax.fori_loop(0, n_ch, zero_chunk, 0)
    plsc.subcore_barrier()

    def do_dir(g_hbm, s_hbm, out_off):
        ebase = sid * EPT

        def chunk(i, c):
            st = ebase + i * K
            pltpu.sync_copy(g_hbm.at[pl.ds(st, K)], gi_v)
            pltpu.sync_copy(s_hbm.at[pl.ds(st, K)], si_v)
            pltpu.async_copy(msg_hbm.at[gi_v], rows_v, sem).wait()
            pltpu.sync_copy(rows_v, acc.at[si_v], add=True)
            return c
        lax.fori_loop(0, NFULL, chunk, 0)
        st = ebase + NFULL * K
        pltpu.sync_copy(g_hbm.at[pl.ds(st, KTAIL)], gi_t)
        pltpu.sync_copy(s_hbm.at[pl.ds(st, KTAIL)], si_t)
        pltpu.async_copy(msg_hbm.at[gi_t], rows_t, sem).wait()
        pltpu.sync_copy(rows_t, acc.at[si_t], add=True)
        plsc.subcore_barrier()

        def out_chunk(i, c):
            off = pl.multiple_of((ch0 + i) * ROW_GRAN, 8)
            pltpu.sync_copy(acc.at[pl.ds(off, ROW_GRAN)],
                            agg_hbm.at[pl.ds(out_off + off, ROW_GRAN)])
            return c
        lax.fori_loop(0, n_ch, out_chunk, 0)

    @pl.when(cid == 0)
    def _():
        do_dir(src_hbm, dst_hbm, N)
    @pl.when(cid == 1)
    def _():
        do_dir(dstp_hbm, src_hbm, 0)


@functools.cache
def _sc_segsum():
    return pl.kernel(
        _sc_body,
        out_type=jax.ShapeDtypeStruct((2 * N, D), jnp.float32),
        mesh=plsc.VectorSubcoreMesh(core_axis_name="c", subcore_axis_name="s"),
        scratch_types=[
            pltpu.VMEM((K,), jnp.int32),
            pltpu.VMEM((K,), jnp.int32),
            pltpu.VMEM((K, D), jnp.float32),
            pltpu.VMEM((KTAIL,), jnp.int32),
            pltpu.VMEM((KTAIL,), jnp.int32),
            pltpu.VMEM((KTAIL, D), jnp.float32),
            pltpu.VMEM((ROW_GRAN, D), jnp.float32),
            pltpu.VMEM_SHARED((N, D), jnp.float32),
            pltpu.SemaphoreType.DMA,
        ],
    )


def _msg_body(x_ref, w1_ref, b1_ref, w2_ref, b2_ref, o_ref):
    h = jnp.maximum(
        jnp.dot(x_ref[...], w1_ref[0], preferred_element_type=jnp.float32)
        + b1_ref[0], 0.0)
    o_ref[...] = (jnp.dot(h, w2_ref[0], preferred_element_type=jnp.float32)
                  + b2_ref[0])


_W_SPEC = pl.BlockSpec((1, D, D), lambda i: (i * 2 // G, 0, 0))
_B_SPEC = pl.BlockSpec((1, 1, D), lambda i: (i * 2 // G, 0, 0))
_X_SPEC = pl.BlockSpec((BR, D), lambda i: (i, 0))

_tc_msg = pl.pallas_call(
    _msg_body,
    grid=(G,),
    in_specs=[_X_SPEC, _W_SPEC, _B_SPEC, _W_SPEC, _B_SPEC],
    out_specs=_X_SPEC,
    out_shape=jax.ShapeDtypeStruct((2 * N, D), jnp.float32),
)


def _upd_body(a_ref, x_ref, w1a_ref, w1b_ref, b1_ref, w2_ref, b2_ref, o_ref):
    h = jnp.maximum(
        jnp.dot(a_ref[...], w1a_ref[0], preferred_element_type=jnp.float32)
        + jnp.dot(x_ref[...], w1b_ref[0], preferred_element_type=jnp.float32)
        + b1_ref[0], 0.0)
    o_ref[...] = (jnp.dot(h, w2_ref[0], preferred_element_type=jnp.float32)
                  + b2_ref[0])


_tc_upd = pl.pallas_call(
    _upd_body,
    grid=(G,),
    in_specs=[_X_SPEC, _X_SPEC, _W_SPEC, _W_SPEC, _B_SPEC, _W_SPEC, _B_SPEC],
    out_specs=_X_SPEC,
    out_shape=jax.ShapeDtypeStruct((2 * N, D), jnp.float32),
)


def kernel(v_emb, c_emb, edge_index,
           vmsg_W1, vmsg_b1, vmsg_W2, vmsg_b2,
           cmsg_W1, cmsg_b1, cmsg_W2, cmsg_b2,
           vupd_W1, vupd_b1, vupd_W2, vupd_b2,
           cupd_W1, cupd_b1, cupd_W2, cupd_b2):
    src = edge_index[0].astype(jnp.int32)
    dst = edge_index[1].astype(jnp.int32)
    dstp = dst + N

    msg_W1 = jnp.stack([vmsg_W1, cmsg_W1])
    msg_b1 = jnp.stack([vmsg_b1, cmsg_b1])[:, None, :]
    msg_W2 = jnp.stack([vmsg_W2, cmsg_W2])
    msg_b2 = jnp.stack([vmsg_b2, cmsg_b2])[:, None, :]
    upd_W1a = jnp.stack([vupd_W1[:D], cupd_W1[:D]])
    upd_W1b = jnp.stack([vupd_W1[D:], cupd_W1[D:]])
    upd_b1 = jnp.stack([vupd_b1, cupd_b1])[:, None, :]
    upd_W2 = jnp.stack([vupd_W2, cupd_W2])
    upd_b2 = jnp.stack([vupd_b2, cupd_b2])[:, None, :]

    emb = jnp.concatenate([v_emb, c_emb], axis=0)
    for _ in range(NUM_ROUND):
        msg = _tc_msg(emb, msg_W1, msg_b1, msg_W2, msg_b2)
        agg = _sc_segsum()(msg, src, dst, dstp)
        emb = _tc_upd(agg, emb, upd_W1a, upd_W1b, upd_b1, upd_W2, upd_b2)
    return (emb[:N], emb[N:])


# speedup vs baseline: 8.5464x; 1.5272x over previous
import functools

import jax
import jax.numpy as jnp
from jax import lax
from jax.experimental import pallas as pl
from jax.experimental.pallas import tpu as pltpu
from jax.experimental.pallas import tpu_sc as plsc

NUM_ROUND = 4
N = 10000
E = 320000
D = 128

NS = 16
EPT = E // NS
K = 80
CPT = EPT // K
NHALF = CPT // 2
ROW_GRAN = 80
NCH = N // ROW_GRAN
CH_BASE = NCH // NS
CH_EXTRA = NCH % NS

BR = 2000
G = (2 * N) // BR


def _sc_body(msg_hbm, src_hbm, dst_hbm, dstp_hbm,
             agg_hbm,
             gidxA, gidxB, sidxA, sidxB, rows0, rows1, acc,
             igsemA, issemA, igsemB, issemB, gsemA, gsemB):
    cid = lax.axis_index("c")
    sid = lax.axis_index("s")
    n_ch = CH_BASE + jnp.where(sid < CH_EXTRA, 1, 0)
    ch0 = sid * CH_BASE + jnp.minimum(sid, CH_EXTRA)

    def zrow(i, c):
        for j in range(8):
            rows0[i, pl.ds(j * 16, 16)] = jnp.zeros((16,), jnp.float32)
        return c
    lax.fori_loop(0, ROW_GRAN, zrow, 0)

    def zero_chunk(i, c):
        off = pl.multiple_of((ch0 + i) * ROW_GRAN, 8)
        pltpu.sync_copy(rows0, acc.at[pl.ds(off, ROW_GRAN)])
        return c
    lax.fori_loop(0, n_ch, zero_chunk, 0)
    plsc.subcore_barrier()

    def do_dir(g_hbm, s_hbm, out_off):
        ebase = sid * EPT

        def istart(gidx, sidx, igsem, issem, i):
            st = ebase + i * K
            pltpu.async_copy(g_hbm.at[pl.ds(st, K)], gidx, igsem)
            pltpu.async_copy(s_hbm.at[pl.ds(st, K)], sidx, issem)

        def iwait(gidx, sidx, igsem, issem):
            pltpu.make_async_copy(g_hbm.at[pl.ds(0, K)], gidx, igsem).wait()
            pltpu.make_async_copy(s_hbm.at[pl.ds(0, K)], sidx, issem).wait()

        def gstart(gidx, rows, gsem):
            pltpu.async_copy(msg_hbm.at[gidx], rows, gsem)

        def gwait(gidx, rows, gsem):
            pltpu.make_async_copy(msg_hbm.at[gidx], rows, gsem).wait()

        def sstore(sidx, rows):
            pltpu.sync_copy(rows, acc.at[sidx], add=True)

        istart(gidxA, sidxA, igsemA, issemA, 0)
        istart(gidxB, sidxB, igsemB, issemB, 1)
        iwait(gidxA, sidxA, igsemA, issemA)
        gstart(gidxA, rows0, gsemA)

        def pair(k, c):
            c2a = jnp.minimum(2 * k + 2, CPT - 1)
            c2b = jnp.minimum(2 * k + 3, CPT - 1)
            iwait(gidxB, sidxB, igsemB, issemB)
            gwait(gidxA, rows0, gsemA)
            gstart(gidxB, rows1, gsemB)
            sstore(sidxA, rows0)
            istart(gidxA, sidxA, igsemA, issemA, c2a)
            iwait(gidxA, sidxA, igsemA, issemA)
            gwait(gidxB, rows1, gsemB)
            gstart(gidxA, rows0, gsemA)
            sstore(sidxB, rows1)
            istart(gidxB, sidxB, igsemB, issemB, c2b)
            return c
        lax.fori_loop(0, NHALF, pair, 0)
        gwait(gidxA, rows0, gsemA)
        iwait(gidxB, sidxB, igsemB, issemB)
        plsc.subcore_barrier()

        def out_chunk(i, c):
            off = pl.multiple_of((ch0 + i) * ROW_GRAN, 8)
            pltpu.sync_copy(acc.at[pl.ds(off, ROW_GRAN)],
                            agg_hbm.at[pl.ds(out_off + off, ROW_GRAN)])
            return c
        lax.fori_loop(0, n_ch, out_chunk, 0)

    @pl.when(cid == 0)
    def _():
        do_dir(src_hbm, dst_hbm, N)
    @pl.when(cid == 1)
    def _():
        do_dir(dstp_hbm, src_hbm, 0)


@functools.cache
def _sc_segsum():
    return pl.kernel(
        _sc_body,
        out_type=jax.ShapeDtypeStruct((2 * N, D), jnp.float32),
        mesh=plsc.VectorSubcoreMesh(core_axis_name="c", subcore_axis_name="s"),
        scratch_types=[
            pltpu.VMEM((K,), jnp.int32),
            pltpu.VMEM((K,), jnp.int32),
            pltpu.VMEM((K,), jnp.int32),
            pltpu.VMEM((K,), jnp.int32),
            pltpu.VMEM((K, D), jnp.float32),
            pltpu.VMEM((K, D), jnp.float32),
            pltpu.VMEM_SHARED((N, D), jnp.float32),
            pltpu.SemaphoreType.DMA,
            pltpu.SemaphoreType.DMA,
            pltpu.SemaphoreType.DMA,
            pltpu.SemaphoreType.DMA,
            pltpu.SemaphoreType.DMA,
            pltpu.SemaphoreType.DMA,
        ],
    )


def _msg_body(x_ref, w1_ref, b1_ref, w2_ref, b2_ref, o_ref):
    h = jnp.maximum(
        jnp.dot(x_ref[...], w1_ref[0], preferred_element_type=jnp.float32)
        + b1_ref[0], 0.0)
    o_ref[...] = (jnp.dot(h, w2_ref[0], preferred_element_type=jnp.float32)
                  + b2_ref[0])


_W_SPEC = pl.BlockSpec((1, D, D), lambda i: (i * 2 // G, 0, 0))
_B_SPEC = pl.BlockSpec((1, 1, D), lambda i: (i * 2 // G, 0, 0))
_X_SPEC = pl.BlockSpec((BR, D), lambda i: (i, 0))

_tc_msg = pl.pallas_call(
    _msg_body,
    grid=(G,),
    in_specs=[_X_SPEC, _W_SPEC, _B_SPEC, _W_SPEC, _B_SPEC],
    out_specs=_X_SPEC,
    out_shape=jax.ShapeDtypeStruct((2 * N, D), jnp.float32),
)


def _upd_body(a_ref, x_ref, w1a_ref, w1b_ref, b1_ref, w2_ref, b2_ref, o_ref):
    h = jnp.maximum(
        jnp.dot(a_ref[...], w1a_ref[0], preferred_element_type=jnp.float32)
        + jnp.dot(x_ref[...], w1b_ref[0], preferred_element_type=jnp.float32)
        + b1_ref[0], 0.0)
    o_ref[...] = (jnp.dot(h, w2_ref[0], preferred_element_type=jnp.float32)
                  + b2_ref[0])


_tc_upd = pl.pallas_call(
    _upd_body,
    grid=(G,),
    in_specs=[_X_SPEC, _X_SPEC, _W_SPEC, _W_SPEC, _B_SPEC, _W_SPEC, _B_SPEC],
    out_specs=_X_SPEC,
    out_shape=jax.ShapeDtypeStruct((2 * N, D), jnp.float32),
)


def kernel(v_emb, c_emb, edge_index,
           vmsg_W1, vmsg_b1, vmsg_W2, vmsg_b2,
           cmsg_W1, cmsg_b1, cmsg_W2, cmsg_b2,
           vupd_W1, vupd_b1, vupd_W2, vupd_b2,
           cupd_W1, cupd_b1, cupd_W2, cupd_b2):
    src = edge_index[0].astype(jnp.int32)
    dst = edge_index[1].astype(jnp.int32)
    dstp = dst + N

    msg_W1 = jnp.stack([vmsg_W1, cmsg_W1])
    msg_b1 = jnp.stack([vmsg_b1, cmsg_b1])[:, None, :]
    msg_W2 = jnp.stack([vmsg_W2, cmsg_W2])
    msg_b2 = jnp.stack([vmsg_b2, cmsg_b2])[:, None, :]
    upd_W1a = jnp.stack([vupd_W1[:D], cupd_W1[:D]])
    upd_W1b = jnp.stack([vupd_W1[D:], cupd_W1[D:]])
    upd_b1 = jnp.stack([vupd_b1, cupd_b1])[:, None, :]
    upd_W2 = jnp.stack([vupd_W2, cupd_W2])
    upd_b2 = jnp.stack([vupd_b2, cupd_b2])[:, None, :]

    emb = jnp.concatenate([v_emb, c_emb], axis=0)
    for _ in range(NUM_ROUND):
        msg = _tc_msg(emb, msg_W1, msg_b1, msg_W2, msg_b2)
        agg = _sc_segsum()(msg, src, dst, dstp)
        emb = _tc_upd(agg, emb, upd_W1a, upd_W1b, upd_b1, upd_W2, upd_b2)
    return (emb[:N], emb[N:])
